# baseline (speedup 1.0000x reference)
import functools

import jax
import jax.numpy as jnp
from jax import lax
from jax.experimental import pallas as pl
from jax.experimental.pallas import tpu as pltpu

N = 10000
E = 320000
D_IN = 128
HID = 128
H = 8
DH = HID // H
L = 2
NODE_TYPES = ('author', 'paper')
EDGE_TYPES = (('writes', 'author', 'paper'), ('written_by', 'paper', 'author'), ('cites', 'paper', 'paper'))

_ROW_BLK = 400


def _proj_body(x_ref, w_ref, b_ref, o_ref):
    o_ref[...] = jnp.dot(x_ref[...], w_ref[...],
                         preferred_element_type=jnp.float32) + b_ref[...]


def _proj(x, w, b):
    m, k = x.shape
    n = w.shape[1]
    grid = m // _ROW_BLK
    return pl.pallas_call(
        _proj_body,
        grid=(grid,),
        in_specs=[
            pl.BlockSpec((_ROW_BLK, k), lambda i: (i, 0)),
            pl.BlockSpec((k, n), lambda i: (0, 0)),
            pl.BlockSpec((1, n), lambda i: (0, 0)),
        ],
        out_specs=pl.BlockSpec((_ROW_BLK, n), lambda i: (i, 0)),
        out_shape=jax.ShapeDtypeStruct((m, n), jnp.float32),
    )(x, w, b.reshape(1, n))


def _segment_softmax(alpha, seg, num_segments):
    amax = jax.ops.segment_max(lax.stop_gradient(alpha), seg, num_segments=num_segments)
    amax = jnp.where(jnp.isfinite(amax), amax, 0.0)
    ex = jnp.exp(alpha - amax[seg])
    denom = jax.ops.segment_sum(ex, seg, num_segments=num_segments)
    return ex / (denom[seg] + 1e-16)


def _han_layer(x_dict, ei_dict, params, l):
    xp = {nt: _proj(x_dict[nt], params[f'proj_W_{nt}_{l}'],
                    params[f'proj_b_{nt}_{l}']).reshape(-1, H, DH)
          for nt in NODE_TYPES}
    out_lists = {nt: [] for nt in NODE_TYPES}
    for et, src_t, dst_t in EDGE_TYPES:
        ei = ei_dict[et]
        a_src = (xp[src_t] * params[f'att_src_{et}_{l}']).sum(-1)
        a_dst = (xp[dst_t] * params[f'att_dst_{et}_{l}']).sum(-1)
        s, d = ei[0], ei[1]
        alpha = jax.nn.leaky_relu(a_src[s] + a_dst[d], 0.2)
        alpha = _segment_softmax(alpha, d, N)
        msg = xp[src_t][s] * alpha[:, :, None]
        out = jax.ops.segment_sum(msg, d, num_segments=N).reshape(N, HID)
        out_lists[dst_t].append(jax.nn.relu(out))
    new_x = {}
    for nt in NODE_TYPES:
        outs = jnp.stack(out_lists[nt])
        kx = jnp.tanh(outs @ params[f'k_lin_W_{l}'] + params[f'k_lin_b_{l}']).mean(axis=1)
        score = (params[f'q_{l}'] * kx).sum(-1)
        attn = jax.nn.softmax(score)
        new_x[nt] = (attn[:, None, None] * outs).sum(0)
    return new_x


def kernel(x_author, x_paper, edge_index_writes, edge_index_written_by, edge_index_cites, params):
    x = {'author': x_author, 'paper': x_paper}
    ei = {'writes': edge_index_writes, 'written_by': edge_index_written_by, 'cites': edge_index_cites}
    for l in range(L):
        x = _han_layer(x, ei, params, l)
        if l < L - 1:
            x = {k: jax.nn.relu(v) for k, v in x.items()}
    return (x['author'], x['paper'])


# speedup vs baseline: 1.0834x; 1.0834x over previous
import functools

import jax
import jax.numpy as jnp
from jax import lax
from jax.experimental import pallas as pl
from jax.experimental.pallas import tpu as pltpu
from jax.experimental.pallas import tpu_sc as plsc

N = 10000
E = 320000
D_IN = 128
HID = 128
H = 8
DH = HID // H
L = 2
NODE_TYPES = ('author', 'paper')
EDGE_TYPES = (('writes', 'author', 'paper'), ('written_by', 'paper', 'author'), ('cites', 'paper', 'paper'))

_ROW_BLK = 400

_CHUNK = 128
_NCHUNK = E // _CHUNK
_NTILES = 32
_ROWS_PER_TILE = 632


def _proj_body(x_ref, w_ref, b_ref, o_ref):
    o_ref[...] = jnp.dot(x_ref[...], w_ref[...],
                         preferred_element_type=jnp.float32) + b_ref[...]


def _proj(x, w, b):
    m, k = x.shape
    n = w.shape[1]
    grid = m // _ROW_BLK
    return pl.pallas_call(
        _proj_body,
        grid=(grid,),
        in_specs=[
            pl.BlockSpec((_ROW_BLK, k), lambda i: (i, 0)),
            pl.BlockSpec((k, n), lambda i: (0, 0)),
            pl.BlockSpec((1, n), lambda i: (0, 0)),
        ],
        out_specs=pl.BlockSpec((_ROW_BLK, n), lambda i: (i, 0)),
        out_shape=jax.ShapeDtypeStruct((m, n), jnp.float32),
    )(x, w, b.reshape(1, n))


def _lrelu(v):
    return jnp.where(v >= 0.0, v, 0.2 * v)


_N8 = N * H
_DEN_PER_TILE = _N8 // 16


def _pass_a_body(asrc_w, adst_w, asrc_wb, adst_wb, asrc_c, adst_c,
                 mx_hbm, zeros_hbm,
                 s_w, d_w, s_wb, d_wb, s_c, d_c,
                 ex_w, ex_wb, ex_c, dp_w, dp_wb, dp_c,
                 sidx_v, didx_v, sidxh_v, didxh_v, sg_v, dg_v, exh_v, mx_v,
                 stage_v, dsp_w, dsp_wb, dsp_c, sem):
    cid_ax = lax.axis_index("c")
    sid_ax = lax.axis_index("s")
    wid = sid_ax * 2 + cid_ax
    dbase = sid_ax * _DEN_PER_TILE

    pltpu.sync_copy(zeros_hbm.at[pl.ds(dbase, _DEN_PER_TILE)], stage_v)
    for dsp in (dsp_w, dsp_wb, dsp_c):
        pltpu.sync_copy(stage_v, dsp.at[pl.ds(dbase, _DEN_PER_TILE)])
    plsc.subcore_barrier()

    pltpu.sync_copy(mx_hbm, mx_v)

    for et, (asrc, adst, s_e, d_e, ex_hbm, dsp) in enumerate((
            (asrc_w, adst_w, s_w, d_w, ex_w, dsp_w),
            (asrc_wb, adst_wb, s_wb, d_wb, ex_wb, dsp_wb),
            (asrc_c, adst_c, s_c, d_c, ex_c, dsp_c))):
        c_et = mx_v[pl.ds(et * 16, 16)]

        def chunk_body(k, _, asrc=asrc, adst=adst, s_e=s_e, d_e=d_e,
                       ex_hbm=ex_hbm, dsp=dsp, c_et=c_et):
            cid = wid + _NTILES * k

            @pl.when(cid < _NCHUNK)
            def _():
                base = cid * _CHUNK
                pltpu.sync_copy(s_e.at[pl.ds(base, _CHUNK)], sidx_v)
                pltpu.sync_copy(d_e.at[pl.ds(base, _CHUNK)], didx_v)
                for i in range(_CHUNK // 16):
                    s16 = sidx_v[pl.ds(i * 16, 16)]
                    d16 = didx_v[pl.ds(i * 16, 16)]
                    for h in range(H):
                        sidxh_v[h, pl.ds(i * 16, 16)] = s16 + (h * N)
                        didxh_v[h, pl.ds(i * 16, 16)] = d16 + (h * N)
                cps = []
                for h in range(H):
                    cps.append(pltpu.async_copy(
                        asrc.at[sidxh_v.at[h]], sg_v.at[h], sem))
                    cps.append(pltpu.async_copy(
                        adst.at[didxh_v.at[h]], dg_v.at[h], sem))
                for cp in cps:
                    cp.wait()
                for h in range(H):
                    for i in range(_CHUNK // 16):
                        v = (sg_v[h, pl.ds(i * 16, 16)]
                             + dg_v[h, pl.ds(i * 16, 16)])
                        exh_v[h, pl.ds(i * 16, 16)] = jnp.exp(_lrelu(v) - c_et)
                for h in range(H):
                    pltpu.sync_copy(exh_v.at[h],
                                    ex_hbm.at[pl.ds(h * E + base, _CHUNK)])
                    pltpu.sync_copy(exh_v.at[h], dsp.at[didxh_v.at[h]],
                                    add=True)
            return 0

        lax.fori_loop(0, (_NCHUNK + _NTILES - 1) // _NTILES, chunk_body, 0)

    plsc.subcore_barrier()
    for dsp, dp in ((dsp_w, dp_w), (dsp_wb, dp_wb), (dsp_c, dp_c)):
        pltpu.sync_copy(dsp.at[pl.ds(dbase, _DEN_PER_TILE)], stage_v)
        pltpu.sync_copy(stage_v,
                        dp.at[pl.ds(cid_ax * _N8 + dbase, _DEN_PER_TILE)])


@jax.jit
def _pass_a(asrc_w, adst_w, asrc_wb, adst_wb, asrc_c, adst_c, mx, zeros,
            s_w, d_w, s_wb, d_wb, s_c, d_c):
    mesh = plsc.VectorSubcoreMesh(core_axis_name="c", subcore_axis_name="s")
    f = pl.kernel(
        _pass_a_body,
        mesh=mesh,
        out_type=[jax.ShapeDtypeStruct((H * E,), jnp.float32)] * 3
                 + [jax.ShapeDtypeStruct((2 * _N8,), jnp.float32)] * 3,
        scratch_types=[
            pltpu.VMEM((_CHUNK,), jnp.int32),
            pltpu.VMEM((_CHUNK,), jnp.int32),
            pltpu.VMEM((H, _CHUNK), jnp.int32),
            pltpu.VMEM((H, _CHUNK), jnp.int32),
            pltpu.VMEM((H, _CHUNK), jnp.float32),
            pltpu.VMEM((H, _CHUNK), jnp.float32),
            pltpu.VMEM((H, _CHUNK), jnp.float32),
            pltpu.VMEM((48,), jnp.float32),
            pltpu.VMEM((_DEN_PER_TILE,), jnp.float32),
            pltpu.VMEM_SHARED((_N8,), jnp.float32),
            pltpu.VMEM_SHARED((_N8,), jnp.float32),
            pltpu.VMEM_SHARED((_N8,), jnp.float32),
            pltpu.SemaphoreType.DMA,
        ],
    )
    return f(asrc_w, adst_w, asrc_wb, adst_wb, asrc_c, adst_c, mx, zeros,
             s_w, d_w, s_wb, d_wb, s_c, d_c)


def _han_layer(x_dict, ei_dict, params, l):
    xp = {nt: _proj(x_dict[nt], params[f'proj_W_{nt}_{l}'],
                    params[f'proj_b_{nt}_{l}']).reshape(-1, H, DH)
          for nt in NODE_TYPES}

    a_src = {}
    a_dst = {}
    mx_rows = []
    for et, src_t, dst_t in EDGE_TYPES:
        asrc = (xp[src_t] * params[f'att_src_{et}_{l}']).sum(-1)
        adst = (xp[dst_t] * params[f'att_dst_{et}_{l}']).sum(-1)
        a_src[et] = asrc.T.reshape(-1)
        a_dst[et] = adst.T.reshape(-1)
        c_et = _lrelu(jnp.max(asrc) + jnp.max(adst))
        mx_rows.append(jnp.full((16,), c_et, jnp.float32))
    mx = jnp.concatenate(mx_rows)
    zeros = jnp.zeros((_N8,), jnp.float32)

    s = {et: ei_dict[et][0] for et, _, _ in EDGE_TYPES}
    d = {et: ei_dict[et][1] for et, _, _ in EDGE_TYPES}

    ex_w, ex_wb, ex_c, dp_w, dp_wb, dp_c = _pass_a(
        a_src['writes'], a_dst['writes'], a_src['written_by'],
        a_dst['written_by'], a_src['cites'], a_dst['cites'], mx, zeros,
        s['writes'], d['writes'], s['written_by'], d['written_by'],
        s['cites'], d['cites'])
    ex = {'writes': ex_w.reshape(H, E).T, 'written_by': ex_wb.reshape(H, E).T,
          'cites': ex_c.reshape(H, E).T}
    denom = {
        'writes': (dp_w[:_N8] + dp_w[_N8:]).reshape(H, N).T,
        'written_by': (dp_wb[:_N8] + dp_wb[_N8:]).reshape(H, N).T,
        'cites': (dp_c[:_N8] + dp_c[_N8:]).reshape(H, N).T,
    }

    out_lists = {nt: [] for nt in NODE_TYPES}
    for et, src_t, dst_t in EDGE_TYPES:
        alpha = ex[et] / (denom[et][d[et]] + 1e-16)
        msg = xp[src_t][s[et]] * alpha[:, :, None]
        out = jax.ops.segment_sum(msg, d[et], num_segments=N).reshape(N, HID)
        out_lists[dst_t].append(jax.nn.relu(out))

    new_x = {}
    for nt in NODE_TYPES:
        outs = jnp.stack(out_lists[nt])
        kx = jnp.tanh(outs @ params[f'k_lin_W_{l}'] + params[f'k_lin_b_{l}']).mean(axis=1)
        score = (params[f'q_{l}'] * kx).sum(-1)
        attn = jax.nn.softmax(score)
        new_x[nt] = (attn[:, None, None] * outs).sum(0)
    return new_x


def kernel(x_author, x_paper, edge_index_writes, edge_index_written_by, edge_index_cites, params):
    x = {'author': x_author, 'paper': x_paper}
    ei = {'writes': edge_index_writes, 'written_by': edge_index_written_by, 'cites': edge_index_cites}
    for l in range(L):
        x = _han_layer(x, ei, params, l)
        if l < L - 1:
            x = {k: jax.nn.relu(v) for k, v in x.items()}
    return (x['author'], x['paper'])


# speedup vs baseline: 26.8544x; 24.7879x over previous
import functools

import jax
import jax.numpy as jnp
from jax import lax
from jax.experimental import pallas as pl
from jax.experimental.pallas import tpu as pltpu
from jax.experimental.pallas import tpu_sc as plsc

N = 10000
E = 320000
D_IN = 128
HID = 128
H = 8
DH = HID // H
L = 2
NODE_TYPES = ('author', 'paper')
EDGE_TYPES = (('writes', 'author', 'paper'), ('written_by', 'paper', 'author'), ('cites', 'paper', 'paper'))

_ROW_BLK = 400

_CHUNK = 128
_NCHUNK = E // _CHUNK
_NTILES = 32
_ROWS_PER_TILE = 632


def _proj_body(x_ref, w_ref, b_ref, o_ref):
    o_ref[...] = jnp.dot(x_ref[...], w_ref[...],
                         preferred_element_type=jnp.float32) + b_ref[...]


def _proj(x, w, b):
    m, k = x.shape
    n = w.shape[1]
    grid = m // _ROW_BLK
    return pl.pallas_call(
        _proj_body,
        grid=(grid,),
        in_specs=[
            pl.BlockSpec((_ROW_BLK, k), lambda i: (i, 0)),
            pl.BlockSpec((k, n), lambda i: (0, 0)),
            pl.BlockSpec((1, n), lambda i: (0, 0)),
        ],
        out_specs=pl.BlockSpec((_ROW_BLK, n), lambda i: (i, 0)),
        out_shape=jax.ShapeDtypeStruct((m, n), jnp.float32),
    )(x, w, b.reshape(1, n))


def _lrelu(v):
    return jnp.where(v >= 0.0, v, 0.2 * v)


_N8 = N * H
_DEN_PER_TILE = _N8 // 16


def _pass_a_body(asrc_w, adst_w, asrc_wb, adst_wb, asrc_c, adst_c,
                 mx_hbm, zeros_hbm,
                 s_w, d_w, s_wb, d_wb, s_c, d_c,
                 ex_w, ex_wb, ex_c, dp_w, dp_wb, dp_c,
                 sidx_v, didx_v, sidxh_v, didxh_v, sg_v, dg_v, exh_v, mx_v,
                 stage_v, dsp_w, dsp_wb, dsp_c, sem):
    cid_ax = lax.axis_index("c")
    sid_ax = lax.axis_index("s")
    wid = sid_ax * 2 + cid_ax
    dbase = sid_ax * _DEN_PER_TILE

    pltpu.sync_copy(zeros_hbm.at[pl.ds(dbase, _DEN_PER_TILE)], stage_v)
    for dsp in (dsp_w, dsp_wb, dsp_c):
        pltpu.sync_copy(stage_v, dsp.at[pl.ds(dbase, _DEN_PER_TILE)])
    plsc.subcore_barrier()

    pltpu.sync_copy(mx_hbm, mx_v)

    for et, (asrc, adst, s_e, d_e, ex_hbm, dsp) in enumerate((
            (asrc_w, adst_w, s_w, d_w, ex_w, dsp_w),
            (asrc_wb, adst_wb, s_wb, d_wb, ex_wb, dsp_wb),
            (asrc_c, adst_c, s_c, d_c, ex_c, dsp_c))):
        c_et = mx_v[pl.ds(et * 16, 16)]

        def chunk_body(k, _, asrc=asrc, adst=adst, s_e=s_e, d_e=d_e,
                       ex_hbm=ex_hbm, dsp=dsp, c_et=c_et):
            cid = wid + _NTILES * k

            @pl.when(cid < _NCHUNK)
            def _():
                base = cid * _CHUNK
                pltpu.sync_copy(s_e.at[pl.ds(base, _CHUNK)], sidx_v)
                pltpu.sync_copy(d_e.at[pl.ds(base, _CHUNK)], didx_v)
                for i in range(_CHUNK // 16):
                    s16 = sidx_v[pl.ds(i * 16, 16)]
                    d16 = didx_v[pl.ds(i * 16, 16)]
                    for h in range(H):
                        sidxh_v[h, pl.ds(i * 16, 16)] = s16 + (h * N)
                        didxh_v[h, pl.ds(i * 16, 16)] = d16 + (h * N)
                cps = []
                for h in range(H):
                    cps.append(pltpu.async_copy(
                        asrc.at[sidxh_v.at[h]], sg_v.at[h], sem))
                    cps.append(pltpu.async_copy(
                        adst.at[didxh_v.at[h]], dg_v.at[h], sem))
                for cp in cps:
                    cp.wait()
                for h in range(H):
                    for i in range(_CHUNK // 16):
                        v = (sg_v[h, pl.ds(i * 16, 16)]
                             + dg_v[h, pl.ds(i * 16, 16)])
                        exh_v[h, pl.ds(i * 16, 16)] = jnp.exp(_lrelu(v) - c_et)
                for h in range(H):
                    pltpu.sync_copy(exh_v.at[h],
                                    ex_hbm.at[pl.ds(h * E + base, _CHUNK)])
                    pltpu.sync_copy(exh_v.at[h], dsp.at[didxh_v.at[h]],
                                    add=True)
            return 0

        lax.fori_loop(0, (_NCHUNK + _NTILES - 1) // _NTILES, chunk_body, 0)

    plsc.subcore_barrier()
    for dsp, dp in ((dsp_w, dp_w), (dsp_wb, dp_wb), (dsp_c, dp_c)):
        pltpu.sync_copy(dsp.at[pl.ds(dbase, _DEN_PER_TILE)], stage_v)
        pltpu.sync_copy(stage_v,
                        dp.at[pl.ds(cid_ax * _N8 + dbase, _DEN_PER_TILE)])


@jax.jit
def _pass_a(asrc_w, adst_w, asrc_wb, adst_wb, asrc_c, adst_c, mx, zeros,
            s_w, d_w, s_wb, d_wb, s_c, d_c):
    mesh = plsc.VectorSubcoreMesh(core_axis_name="c", subcore_axis_name="s")
    f = pl.kernel(
        _pass_a_body,
        mesh=mesh,
        out_type=[jax.ShapeDtypeStruct((H * E,), jnp.float32)] * 3
                 + [jax.ShapeDtypeStruct((2 * _N8,), jnp.float32)] * 3,
        scratch_types=[
            pltpu.VMEM((_CHUNK,), jnp.int32),
            pltpu.VMEM((_CHUNK,), jnp.int32),
            pltpu.VMEM((H, _CHUNK), jnp.int32),
            pltpu.VMEM((H, _CHUNK), jnp.int32),
            pltpu.VMEM((H, _CHUNK), jnp.float32),
            pltpu.VMEM((H, _CHUNK), jnp.float32),
            pltpu.VMEM((H, _CHUNK), jnp.float32),
            pltpu.VMEM((48,), jnp.float32),
            pltpu.VMEM((_DEN_PER_TILE,), jnp.float32),
            pltpu.VMEM_SHARED((_N8,), jnp.float32),
            pltpu.VMEM_SHARED((_N8,), jnp.float32),
            pltpu.VMEM_SHARED((_N8,), jnp.float32),
            pltpu.SemaphoreType.DMA,
        ],
    )
    return f(asrc_w, adst_w, asrc_wb, adst_wb, asrc_c, adst_c, mx, zeros,
             s_w, d_w, s_wb, d_wb, s_c, d_c)


_HALF = N // 2
_OSP_ROWS = _HALF + 8
_ORPT = 320


def _bcast_lane(v, lane):
    idx = jnp.full((16, 1), lane, jnp.int32)
    dn = lax.GatherDimensionNumbers(offset_dims=(), collapsed_slice_dims=(0,),
                                    start_index_map=(0,))
    return lax.gather(v, idx, dn, (1,),
                      mode=lax.GatherScatterMode.PROMISE_IN_BOUNDS)


def _pass_b_body(xp_a, xp_p, exT_w, exT_wb, exT_c, rdT_w, rdT_wb, rdT_c,
                 s_w, d_w, s_wb, d_wb, s_c, d_c, zeros2d_hbm,
                 out_w, out_wb, out_c,
                 sidx_v, didx_v, didxc_v, didxh_v, rd_v, exv_v, w_v, rows_v,
                 zbuf_v, osp, sem, sem2):
    cid_ax = lax.axis_index("c")
    sid_ax = lax.axis_index("s")
    half_base = cid_ax * _HALF
    zrb = jnp.minimum(sid_ax * _ORPT, _OSP_ROWS - _ORPT)
    orb = jnp.minimum(sid_ax * _ORPT, _HALF - _ORPT)

    def zero_osp():
        pltpu.sync_copy(zeros2d_hbm, zbuf_v)
        pltpu.sync_copy(zbuf_v, osp.at[pl.ds(zrb, _ORPT), :])

    zero_osp()
    plsc.subcore_barrier()

    for et, (xp_hbm, exT_hbm, rdT_hbm, s_e, d_e, out_hbm) in enumerate((
            (xp_a, exT_w, rdT_w, s_w, d_w, out_w),
            (xp_p, exT_wb, rdT_wb, s_wb, d_wb, out_wb),
            (xp_p, exT_c, rdT_c, s_c, d_c, out_c))):

        def chunk_body(k, _, xp_hbm=xp_hbm, exT_hbm=exT_hbm, rdT_hbm=rdT_hbm,
                       s_e=s_e, d_e=d_e):
            cid = sid_ax + 16 * k

            @pl.when(cid < _NCHUNK)
            def _():
                base = cid * _CHUNK
                pltpu.sync_copy(s_e.at[pl.ds(base, _CHUNK)], sidx_v)
                pltpu.sync_copy(d_e.at[pl.ds(base, _CHUNK)], didx_v)
                rows_cp = pltpu.async_copy(xp_hbm.at[sidx_v], rows_v, sem2)
                for i in range(_CHUNK // 16):
                    d16 = didx_v[pl.ds(i * 16, 16)]
                    for h in range(H):
                        didxh_v[h, pl.ds(i * 16, 16)] = d16 + (h * N)
                    local = d16 - half_base
                    ok = (local >= 0) & (local < _HALF)
                    didxc_v[pl.ds(i * 16, 16)] = jnp.where(ok, local, _HALF)
                cps = []
                for h in range(H):
                    cps.append(pltpu.async_copy(
                        rdT_hbm.at[didxh_v.at[h]], rd_v.at[h], sem))
                    pltpu.sync_copy(exT_hbm.at[pl.ds(h * E + base, _CHUNK)],
                                    exv_v.at[h])
                for cp in cps:
                    cp.wait()
                for h in range(H):
                    for i in range(_CHUNK // 16):
                        w_v[h, pl.ds(i * 16, 16)] = (
                            exv_v[h, pl.ds(i * 16, 16)]
                            * rd_v[h, pl.ds(i * 16, 16)])
                rows_cp.wait()

                def group_body(g, _):
                    wl = [w_v[h, pl.ds(g * 16, 16)] for h in range(H)]
                    for e in range(16):
                        r = g * 16 + e
                        for h in range(H):
                            bc = _bcast_lane(wl[h], e)
                            rows_v[r, pl.ds(h * 16, 16)] = (
                                rows_v[r, pl.ds(h * 16, 16)] * bc)
                    return 0

                lax.fori_loop(0, _CHUNK // 16, group_body, 0)
                pltpu.sync_copy(rows_v, osp.at[didxc_v], add=True)
            return 0

        lax.fori_loop(0, (_NCHUNK + 15) // 16, chunk_body, 0)

        plsc.subcore_barrier()
        pltpu.sync_copy(osp.at[pl.ds(orb, _ORPT), :], zbuf_v)
        pltpu.sync_copy(zbuf_v, out_hbm.at[pl.ds(half_base + orb, _ORPT), :])
        plsc.subcore_barrier()
        if et < 2:
            zero_osp()
            plsc.subcore_barrier()


@jax.jit
def _pass_b(xp_a, xp_p, exT_w, exT_wb, exT_c, rdT_w, rdT_wb, rdT_c,
            s_w, d_w, s_wb, d_wb, s_c, d_c, zeros2d):
    mesh = plsc.VectorSubcoreMesh(core_axis_name="c", subcore_axis_name="s")
    f = pl.kernel(
        _pass_b_body,
        mesh=mesh,
        out_type=[jax.ShapeDtypeStruct((N, HID), jnp.float32)] * 3,
        scratch_types=[
            pltpu.VMEM((_CHUNK,), jnp.int32),
            pltpu.VMEM((_CHUNK,), jnp.int32),
            pltpu.VMEM((_CHUNK,), jnp.int32),
            pltpu.VMEM((H, _CHUNK), jnp.int32),
            pltpu.VMEM((H, _CHUNK), jnp.float32),
            pltpu.VMEM((H, _CHUNK), jnp.float32),
            pltpu.VMEM((H, _CHUNK), jnp.float32),
            pltpu.VMEM((_CHUNK, HID), jnp.float32),
            pltpu.VMEM((_ORPT, HID), jnp.float32),
            pltpu.VMEM_SHARED((_OSP_ROWS, HID), jnp.float32),
            pltpu.SemaphoreType.DMA,
            pltpu.SemaphoreType.DMA,
        ],
    )
    return f(xp_a, xp_p, exT_w, exT_wb, exT_c, rdT_w, rdT_wb, rdT_c,
             s_w, d_w, s_wb, d_wb, s_c, d_c, zeros2d)


def _han_layer(x_dict, ei_dict, params, l):
    xp = {nt: _proj(x_dict[nt], params[f'proj_W_{nt}_{l}'],
                    params[f'proj_b_{nt}_{l}']).reshape(-1, H, DH)
          for nt in NODE_TYPES}

    a_src = {}
    a_dst = {}
    mx_rows = []
    for et, src_t, dst_t in EDGE_TYPES:
        asrc = (xp[src_t] * params[f'att_src_{et}_{l}']).sum(-1)
        adst = (xp[dst_t] * params[f'att_dst_{et}_{l}']).sum(-1)
        a_src[et] = asrc.T.reshape(-1)
        a_dst[et] = adst.T.reshape(-1)
        c_et = _lrelu(jnp.max(asrc) + jnp.max(adst))
        mx_rows.append(jnp.full((16,), c_et, jnp.float32))
    mx = jnp.concatenate(mx_rows)
    zeros = jnp.zeros((_N8,), jnp.float32)

    s = {et: ei_dict[et][0] for et, _, _ in EDGE_TYPES}
    d = {et: ei_dict[et][1] for et, _, _ in EDGE_TYPES}

    ex_w, ex_wb, ex_c, dp_w, dp_wb, dp_c = _pass_a(
        a_src['writes'], a_dst['writes'], a_src['written_by'],
        a_dst['written_by'], a_src['cites'], a_dst['cites'], mx, zeros,
        s['writes'], d['writes'], s['written_by'], d['written_by'],
        s['cites'], d['cites'])
    exT = {'writes': ex_w, 'written_by': ex_wb, 'cites': ex_c}
    rdT = {
        'writes': 1.0 / (dp_w[:_N8] + dp_w[_N8:] + 1e-16),
        'written_by': 1.0 / (dp_wb[:_N8] + dp_wb[_N8:] + 1e-16),
        'cites': 1.0 / (dp_c[:_N8] + dp_c[_N8:] + 1e-16),
    }

    zeros2d = jnp.zeros((_ORPT, HID), jnp.float32)
    out_w, out_wb, out_c = _pass_b(
        xp['author'].reshape(N, HID), xp['paper'].reshape(N, HID),
        exT['writes'], exT['written_by'], exT['cites'],
        rdT['writes'], rdT['written_by'], rdT['cites'],
        s['writes'], d['writes'], s['written_by'], d['written_by'],
        s['cites'], d['cites'], zeros2d)
    out_lists = {
        'author': [jax.nn.relu(out_wb)],
        'paper': [jax.nn.relu(out_w), jax.nn.relu(out_c)],
    }

    new_x = {}
    for nt in NODE_TYPES:
        outs = jnp.stack(out_lists[nt])
        kx = jnp.tanh(outs @ params[f'k_lin_W_{l}'] + params[f'k_lin_b_{l}']).mean(axis=1)
        score = (params[f'q_{l}'] * kx).sum(-1)
        attn = jax.nn.softmax(score)
        new_x[nt] = (attn[:, None, None] * outs).sum(0)
    return new_x


def kernel(x_author, x_paper, edge_index_writes, edge_index_written_by, edge_index_cites, params):
    x = {'author': x_author, 'paper': x_paper}
    ei = {'writes': edge_index_writes, 'written_by': edge_index_written_by, 'cites': edge_index_cites}
    for l in range(L):
        x = _han_layer(x, ei, params, l)
        if l < L - 1:
            x = {k: jax.nn.relu(v) for k, v in x.items()}
    return (x['author'], x['paper'])


# speedup vs baseline: 39.9746x; 1.4886x over previous
import functools

import jax
import jax.numpy as jnp
from jax import lax
from jax.experimental import pallas as pl
from jax.experimental.pallas import tpu as pltpu
from jax.experimental.pallas import tpu_sc as plsc

N = 10000
E = 320000
D_IN = 128
HID = 128
H = 8
DH = HID // H
L = 2
NODE_TYPES = ('author', 'paper')
EDGE_TYPES = (('writes', 'author', 'paper'), ('written_by', 'paper', 'author'), ('cites', 'paper', 'paper'))

_ROW_BLK = 400

_CHUNK = 128
_NCHUNK = E // _CHUNK
_NTILES = 32
_ROWS_PER_TILE = 632


def _proj_body(x_ref, w_ref, b_ref, o_ref):
    o_ref[...] = jnp.dot(x_ref[...], w_ref[...],
                         preferred_element_type=jnp.float32) + b_ref[...]


def _proj(x, w, b):
    m, k = x.shape
    n = w.shape[1]
    grid = m // _ROW_BLK
    return pl.pallas_call(
        _proj_body,
        grid=(grid,),
        in_specs=[
            pl.BlockSpec((_ROW_BLK, k), lambda i: (i, 0)),
            pl.BlockSpec((k, n), lambda i: (0, 0)),
            pl.BlockSpec((1, n), lambda i: (0, 0)),
        ],
        out_specs=pl.BlockSpec((_ROW_BLK, n), lambda i: (i, 0)),
        out_shape=jax.ShapeDtypeStruct((m, n), jnp.float32),
    )(x, w, b.reshape(1, n))


def _lrelu(v):
    return jnp.where(v >= 0.0, v, 0.2 * v)


_N8 = N * H
_DEN_PER_TILE = _N8 // 16


def _pass_a_body(asrc_w, adst_w, asrc_wb, adst_wb, asrc_c, adst_c,
                 mx_hbm, zeros_hbm,
                 s_w, d_w, s_wb, d_wb, s_c, d_c,
                 ex_w, ex_wb, ex_c, dp_w, dp_wb, dp_c,
                 sidx_v, didx_v, sidxh_v, didxh_v, sg_v, dg_v, exh_v, mx_v,
                 stage_v, dsp_w, dsp_wb, dsp_c, sem):
    cid_ax = lax.axis_index("c")
    sid_ax = lax.axis_index("s")
    wid = sid_ax * 2 + cid_ax
    dbase = sid_ax * _DEN_PER_TILE

    pltpu.sync_copy(zeros_hbm.at[pl.ds(dbase, _DEN_PER_TILE)], stage_v)
    for dsp in (dsp_w, dsp_wb, dsp_c):
        pltpu.sync_copy(stage_v, dsp.at[pl.ds(dbase, _DEN_PER_TILE)])
    plsc.subcore_barrier()

    pltpu.sync_copy(mx_hbm, mx_v)

    for et, (asrc, adst, s_e, d_e, ex_hbm, dsp) in enumerate((
            (asrc_w, adst_w, s_w, d_w, ex_w, dsp_w),
            (asrc_wb, adst_wb, s_wb, d_wb, ex_wb, dsp_wb),
            (asrc_c, adst_c, s_c, d_c, ex_c, dsp_c))):
        c_et = mx_v[pl.ds(et * 16, 16)]

        def chunk_body(k, _, asrc=asrc, adst=adst, s_e=s_e, d_e=d_e,
                       ex_hbm=ex_hbm, dsp=dsp, c_et=c_et):
            cid = wid + _NTILES * k

            @pl.when(cid < _NCHUNK)
            def _():
                base = cid * _CHUNK
                pltpu.sync_copy(s_e.at[pl.ds(base, _CHUNK)], sidx_v)
                pltpu.sync_copy(d_e.at[pl.ds(base, _CHUNK)], didx_v)
                for i in range(_CHUNK // 16):
                    s16 = sidx_v[pl.ds(i * 16, 16)]
                    d16 = didx_v[pl.ds(i * 16, 16)]
                    for h in range(H):
                        sidxh_v[h, pl.ds(i * 16, 16)] = s16 + (h * N)
                        didxh_v[h, pl.ds(i * 16, 16)] = d16 + (h * N)
                cps = []
                for h in range(H):
                    cps.append(pltpu.async_copy(
                        asrc.at[sidxh_v.at[h]], sg_v.at[h], sem))
                    cps.append(pltpu.async_copy(
                        adst.at[didxh_v.at[h]], dg_v.at[h], sem))
                for cp in cps:
                    cp.wait()
                for h in range(H):
                    for i in range(_CHUNK // 16):
                        v = (sg_v[h, pl.ds(i * 16, 16)]
                             + dg_v[h, pl.ds(i * 16, 16)])
                        exh_v[h, pl.ds(i * 16, 16)] = jnp.exp(_lrelu(v) - c_et)
                pltpu.sync_copy(exh_v, ex_hbm.at[cid])
                for h in range(H):
                    pltpu.sync_copy(exh_v.at[h], dsp.at[didxh_v.at[h]],
                                    add=True)
            return 0

        lax.fori_loop(0, (_NCHUNK + _NTILES - 1) // _NTILES, chunk_body, 0)

    plsc.subcore_barrier()
    for dsp, dp in ((dsp_w, dp_w), (dsp_wb, dp_wb), (dsp_c, dp_c)):
        pltpu.sync_copy(dsp.at[pl.ds(dbase, _DEN_PER_TILE)], stage_v)
        pltpu.sync_copy(stage_v,
                        dp.at[pl.ds(cid_ax * _N8 + dbase, _DEN_PER_TILE)])


@jax.jit
def _pass_a(asrc_w, adst_w, asrc_wb, adst_wb, asrc_c, adst_c, mx, zeros,
            s_w, d_w, s_wb, d_wb, s_c, d_c):
    mesh = plsc.VectorSubcoreMesh(core_axis_name="c", subcore_axis_name="s")
    f = pl.kernel(
        _pass_a_body,
        mesh=mesh,
        out_type=[jax.ShapeDtypeStruct((_NCHUNK, H, _CHUNK), jnp.float32)] * 3
                 + [jax.ShapeDtypeStruct((2 * _N8,), jnp.float32)] * 3,
        scratch_types=[
            pltpu.VMEM((_CHUNK,), jnp.int32),
            pltpu.VMEM((_CHUNK,), jnp.int32),
            pltpu.VMEM((H, _CHUNK), jnp.int32),
            pltpu.VMEM((H, _CHUNK), jnp.int32),
            pltpu.VMEM((H, _CHUNK), jnp.float32),
            pltpu.VMEM((H, _CHUNK), jnp.float32),
            pltpu.VMEM((H, _CHUNK), jnp.float32),
            pltpu.VMEM((48,), jnp.float32),
            pltpu.VMEM((_DEN_PER_TILE,), jnp.float32),
            pltpu.VMEM_SHARED((_N8,), jnp.float32),
            pltpu.VMEM_SHARED((_N8,), jnp.float32),
            pltpu.VMEM_SHARED((_N8,), jnp.float32),
            pltpu.SemaphoreType.DMA,
        ],
    )
    return f(asrc_w, adst_w, asrc_wb, adst_wb, asrc_c, adst_c, mx, zeros,
             s_w, d_w, s_wb, d_wb, s_c, d_c)


_HALF = N // 2
_OSP_ROWS = _HALF + 8
_ORPT = 320


def _bcast_lane(v, lane):
    idx = jnp.full((16, 1), lane, jnp.int32)
    dn = lax.GatherDimensionNumbers(offset_dims=(), collapsed_slice_dims=(0,),
                                    start_index_map=(0,))
    return lax.gather(v, idx, dn, (1,),
                      mode=lax.GatherScatterMode.PROMISE_IN_BOUNDS)


def _pass_b_body(xp_a, xp_p, exT_w, exT_wb, exT_c,
                 s_w, d_w, s_wb, d_wb, s_c, d_c, zeros2d_hbm,
                 out_w, out_wb, out_c,
                 sidx_v, didx_v, didxc_v, exv_v, rows_v,
                 zbuf_v, osp, sem, sem2):
    cid_ax = lax.axis_index("c")
    sid_ax = lax.axis_index("s")
    half_base = cid_ax * _HALF
    zrb = jnp.minimum(sid_ax * _ORPT, _OSP_ROWS - _ORPT)
    orb = jnp.minimum(sid_ax * _ORPT, _HALF - _ORPT)

    def zero_osp():
        pltpu.sync_copy(zeros2d_hbm, zbuf_v)
        pltpu.sync_copy(zbuf_v, osp.at[pl.ds(zrb, _ORPT), :])

    zero_osp()
    plsc.subcore_barrier()

    for et, (xp_hbm, exT_hbm, s_e, d_e, out_hbm) in enumerate((
            (xp_a, exT_w, s_w, d_w, out_w),
            (xp_p, exT_wb, s_wb, d_wb, out_wb),
            (xp_p, exT_c, s_c, d_c, out_c))):

        def chunk_body(k, _, xp_hbm=xp_hbm, exT_hbm=exT_hbm,
                       s_e=s_e, d_e=d_e):
            cid = sid_ax + 16 * k

            @pl.when(cid < _NCHUNK)
            def _():
                base = cid * _CHUNK
                pltpu.sync_copy(s_e.at[pl.ds(base, _CHUNK)], sidx_v)
                pltpu.sync_copy(d_e.at[pl.ds(base, _CHUNK)], didx_v)
                rows_cp = pltpu.async_copy(xp_hbm.at[sidx_v], rows_v, sem2)
                ex_cp = pltpu.async_copy(exT_hbm.at[cid], exv_v, sem)
                for i in range(_CHUNK // 16):
                    d16 = didx_v[pl.ds(i * 16, 16)]
                    local = d16 - half_base
                    ok = (local >= 0) & (local < _HALF)
                    didxc_v[pl.ds(i * 16, 16)] = jnp.where(ok, local, _HALF)
                ex_cp.wait()
                rows_cp.wait()

                def group_body(g, _):
                    wl = [exv_v[h, pl.ds(g * 16, 16)] for h in range(H)]
                    for e in range(16):
                        r = g * 16 + e
                        for h in range(H):
                            bc = _bcast_lane(wl[h], e)
                            rows_v[r, pl.ds(h * 16, 16)] = (
                                rows_v[r, pl.ds(h * 16, 16)] * bc)
                    return 0

                lax.fori_loop(0, _CHUNK // 16, group_body, 0)
                pltpu.sync_copy(rows_v, osp.at[didxc_v], add=True)
            return 0

        lax.fori_loop(0, (_NCHUNK + 15) // 16, chunk_body, 0)

        plsc.subcore_barrier()
        pltpu.sync_copy(osp.at[pl.ds(orb, _ORPT), :], zbuf_v)
        pltpu.sync_copy(zbuf_v, out_hbm.at[pl.ds(half_base + orb, _ORPT), :])
        plsc.subcore_barrier()
        if et < 2:
            zero_osp()
            plsc.subcore_barrier()


@jax.jit
def _pass_b(xp_a, xp_p, exT_w, exT_wb, exT_c,
            s_w, d_w, s_wb, d_wb, s_c, d_c, zeros2d):
    mesh = plsc.VectorSubcoreMesh(core_axis_name="c", subcore_axis_name="s")
    f = pl.kernel(
        _pass_b_body,
        mesh=mesh,
        out_type=[jax.ShapeDtypeStruct((N, HID), jnp.float32)] * 3,
        scratch_types=[
            pltpu.VMEM((_CHUNK,), jnp.int32),
            pltpu.VMEM((_CHUNK,), jnp.int32),
            pltpu.VMEM((_CHUNK,), jnp.int32),
            pltpu.VMEM((H, _CHUNK), jnp.float32),
            pltpu.VMEM((_CHUNK, HID), jnp.float32),
            pltpu.VMEM((_ORPT, HID), jnp.float32),
            pltpu.VMEM_SHARED((_OSP_ROWS, HID), jnp.float32),
            pltpu.SemaphoreType.DMA,
            pltpu.SemaphoreType.DMA,
        ],
    )
    return f(xp_a, xp_p, exT_w, exT_wb, exT_c,
             s_w, d_w, s_wb, d_wb, s_c, d_c, zeros2d)


def _han_layer(x_dict, ei_dict, params, l):
    xp = {nt: _proj(x_dict[nt], params[f'proj_W_{nt}_{l}'],
                    params[f'proj_b_{nt}_{l}']).reshape(-1, H, DH)
          for nt in NODE_TYPES}

    a_src = {}
    a_dst = {}
    mx_rows = []
    for et, src_t, dst_t in EDGE_TYPES:
        asrc = (xp[src_t] * params[f'att_src_{et}_{l}']).sum(-1)
        adst = (xp[dst_t] * params[f'att_dst_{et}_{l}']).sum(-1)
        a_src[et] = asrc.T.reshape(-1)
        a_dst[et] = adst.T.reshape(-1)
        c_et = _lrelu(jnp.max(asrc) + jnp.max(adst))
        mx_rows.append(jnp.full((16,), c_et, jnp.float32))
    mx = jnp.concatenate(mx_rows)
    zeros = jnp.zeros((_N8,), jnp.float32)

    s = {et: ei_dict[et][0] for et, _, _ in EDGE_TYPES}
    d = {et: ei_dict[et][1] for et, _, _ in EDGE_TYPES}

    ex_w, ex_wb, ex_c, dp_w, dp_wb, dp_c = _pass_a(
        a_src['writes'], a_dst['writes'], a_src['written_by'],
        a_dst['written_by'], a_src['cites'], a_dst['cites'], mx, zeros,
        s['writes'], d['writes'], s['written_by'], d['written_by'],
        s['cites'], d['cites'])
    def rd_exp(dp):
        rd = 1.0 / (dp[:_N8] + dp[_N8:] + 1e-16)
        return jnp.repeat(rd.reshape(H, N).T, DH, axis=1)

    zeros2d = jnp.zeros((_ORPT, HID), jnp.float32)
    out_w, out_wb, out_c = _pass_b(
        xp['author'].reshape(N, HID), xp['paper'].reshape(N, HID),
        ex_w, ex_wb, ex_c,
        s['writes'], d['writes'], s['written_by'], d['written_by'],
        s['cites'], d['cites'], zeros2d)
    out_lists = {
        'author': [jax.nn.relu(out_wb * rd_exp(dp_wb))],
        'paper': [jax.nn.relu(out_w * rd_exp(dp_w)),
                  jax.nn.relu(out_c * rd_exp(dp_c))],
    }

    new_x = {}
    for nt in NODE_TYPES:
        outs = jnp.stack(out_lists[nt])
        kx = jnp.tanh(outs @ params[f'k_lin_W_{l}'] + params[f'k_lin_b_{l}']).mean(axis=1)
        score = (params[f'q_{l}'] * kx).sum(-1)
        attn = jax.nn.softmax(score)
        new_x[nt] = (attn[:, None, None] * outs).sum(0)
    return new_x


def kernel(x_author, x_paper, edge_index_writes, edge_index_written_by, edge_index_cites, params):
    x = {'author': x_author, 'paper': x_paper}
    ei = {'writes': edge_index_writes, 'written_by': edge_index_written_by, 'cites': edge_index_cites}
    for l in range(L):
        x = _han_layer(x, ei, params, l)
        if l < L - 1:
            x = {k: jax.nn.relu(v) for k, v in x.items()}
    return (x['author'], x['paper'])


# speedup vs baseline: 51.4346x; 1.2867x over previous
import functools

import jax
import jax.numpy as jnp
from jax import lax
from jax.experimental import pallas as pl
from jax.experimental.pallas import tpu as pltpu
from jax.experimental.pallas import tpu_sc as plsc

N = 10000
E = 320000
D_IN = 128
HID = 128
H = 8
DH = HID // H
L = 2
NODE_TYPES = ('author', 'paper')
EDGE_TYPES = (('writes', 'author', 'paper'), ('written_by', 'paper', 'author'), ('cites', 'paper', 'paper'))

_ROW_BLK = 400

_CHUNK = 128
_NCHUNK = E // _CHUNK
_NTILES = 32
_ROWS_PER_TILE = 632


def _proj_body(x_ref, w_ref, b_ref, o_ref):
    o_ref[...] = jnp.dot(x_ref[...], w_ref[...],
                         preferred_element_type=jnp.float32) + b_ref[...]


def _proj(x, w, b):
    m, k = x.shape
    n = w.shape[1]
    grid = m // _ROW_BLK
    return pl.pallas_call(
        _proj_body,
        grid=(grid,),
        in_specs=[
            pl.BlockSpec((_ROW_BLK, k), lambda i: (i, 0)),
            pl.BlockSpec((k, n), lambda i: (0, 0)),
            pl.BlockSpec((1, n), lambda i: (0, 0)),
        ],
        out_specs=pl.BlockSpec((_ROW_BLK, n), lambda i: (i, 0)),
        out_shape=jax.ShapeDtypeStruct((m, n), jnp.float32),
    )(x, w, b.reshape(1, n))


def _lrelu(v):
    return jnp.where(v >= 0.0, v, 0.2 * v)


_N8 = N * H
_DEN_PER_TILE = _N8 // 16


def _pass_a_body(asrc_w, adst_w, asrc_wb, adst_wb, asrc_c, adst_c,
                 mx_hbm, zeros_hbm,
                 s_w, d_w, s_wb, d_wb, s_c, d_c,
                 ex_w, ex_wb, ex_c, dp_w, dp_wb, dp_c,
                 sidx_v, didx_v, sidxh_v, didxh_v, sg_v, dg_v, exh_v, mx_v,
                 stage_v, dsp_w, dsp_wb, dsp_c, sem):
    cid_ax = lax.axis_index("c")
    sid_ax = lax.axis_index("s")
    wid = sid_ax * 2 + cid_ax
    dbase = sid_ax * _DEN_PER_TILE

    pltpu.sync_copy(zeros_hbm.at[pl.ds(dbase, _DEN_PER_TILE)], stage_v)
    for dsp in (dsp_w, dsp_wb, dsp_c):
        pltpu.sync_copy(stage_v, dsp.at[pl.ds(dbase, _DEN_PER_TILE)])
    plsc.subcore_barrier()

    pltpu.sync_copy(mx_hbm, mx_v)

    for et, (asrc, adst, s_e, d_e, ex_hbm, dsp) in enumerate((
            (asrc_w, adst_w, s_w, d_w, ex_w, dsp_w),
            (asrc_wb, adst_wb, s_wb, d_wb, ex_wb, dsp_wb),
            (asrc_c, adst_c, s_c, d_c, ex_c, dsp_c))):
        c_et = mx_v[pl.ds(et * 16, 16)]

        def chunk_body(k, _, asrc=asrc, adst=adst, s_e=s_e, d_e=d_e,
                       ex_hbm=ex_hbm, dsp=dsp, c_et=c_et):
            cid = wid + _NTILES * k

            @pl.when(cid < _NCHUNK)
            def _():
                base = cid * _CHUNK
                pltpu.sync_copy(s_e.at[pl.ds(base, _CHUNK)], sidx_v)
                pltpu.sync_copy(d_e.at[pl.ds(base, _CHUNK)], didx_v)
                for i in range(_CHUNK // 16):
                    s16 = sidx_v[pl.ds(i * 16, 16)]
                    d16 = didx_v[pl.ds(i * 16, 16)]
                    for h in range(H):
                        sidxh_v[h, pl.ds(i * 16, 16)] = s16 + (h * N)
                        didxh_v[h, pl.ds(i * 16, 16)] = d16 + (h * N)
                cps = []
                for h in range(H):
                    cps.append(pltpu.async_copy(
                        asrc.at[sidxh_v.at[h]], sg_v.at[h], sem))
                    cps.append(pltpu.async_copy(
                        adst.at[didxh_v.at[h]], dg_v.at[h], sem))
                for cp in cps:
                    cp.wait()
                for h in range(H):
                    for i in range(_CHUNK // 16):
                        v = (sg_v[h, pl.ds(i * 16, 16)]
                             + dg_v[h, pl.ds(i * 16, 16)])
                        exh_v[h, pl.ds(i * 16, 16)] = jnp.exp(_lrelu(v) - c_et)
                pltpu.sync_copy(exh_v, ex_hbm.at[cid])
                for h in range(H):
                    pltpu.sync_copy(exh_v.at[h], dsp.at[didxh_v.at[h]],
                                    add=True)
            return 0

        lax.fori_loop(0, (_NCHUNK + _NTILES - 1) // _NTILES, chunk_body, 0)

    plsc.subcore_barrier()
    for dsp, dp in ((dsp_w, dp_w), (dsp_wb, dp_wb), (dsp_c, dp_c)):
        pltpu.sync_copy(dsp.at[pl.ds(dbase, _DEN_PER_TILE)], stage_v)
        pltpu.sync_copy(stage_v,
                        dp.at[pl.ds(cid_ax * _N8 + dbase, _DEN_PER_TILE)])


@jax.jit
def _pass_a(asrc_w, adst_w, asrc_wb, adst_wb, asrc_c, adst_c, mx, zeros,
            s_w, d_w, s_wb, d_wb, s_c, d_c):
    mesh = plsc.VectorSubcoreMesh(core_axis_name="c", subcore_axis_name="s")
    f = pl.kernel(
        _pass_a_body,
        mesh=mesh,
        out_type=[jax.ShapeDtypeStruct((_NCHUNK, H, _CHUNK), jnp.float32)] * 3
                 + [jax.ShapeDtypeStruct((2 * _N8,), jnp.float32)] * 3,
        scratch_types=[
            pltpu.VMEM((_CHUNK,), jnp.int32),
            pltpu.VMEM((_CHUNK,), jnp.int32),
            pltpu.VMEM((H, _CHUNK), jnp.int32),
            pltpu.VMEM((H, _CHUNK), jnp.int32),
            pltpu.VMEM((H, _CHUNK), jnp.float32),
            pltpu.VMEM((H, _CHUNK), jnp.float32),
            pltpu.VMEM((H, _CHUNK), jnp.float32),
            pltpu.VMEM((48,), jnp.float32),
            pltpu.VMEM((_DEN_PER_TILE,), jnp.float32),
            pltpu.VMEM_SHARED((_N8,), jnp.float32),
            pltpu.VMEM_SHARED((_N8,), jnp.float32),
            pltpu.VMEM_SHARED((_N8,), jnp.float32),
            pltpu.SemaphoreType.DMA,
        ],
    )
    return f(asrc_w, adst_w, asrc_wb, adst_wb, asrc_c, adst_c, mx, zeros,
             s_w, d_w, s_wb, d_wb, s_c, d_c)


_HALF = N // 2
_OSP_ROWS = _HALF + 8
_ORPT = 320
_SUP = 10
_SUPSZ = _SUP * _CHUNK
_NSUP = E // _SUPSZ


def _bcast_lane(v, lane):
    idx = jnp.full((16, 1), lane, jnp.int32)
    dn = lax.GatherDimensionNumbers(offset_dims=(), collapsed_slice_dims=(0,),
                                    start_index_map=(0,))
    return lax.gather(v, idx, dn, (1,),
                      mode=lax.GatherScatterMode.PROMISE_IN_BOUNDS)


def _pass_b_body(xp_a, xp_p, exT_w, exT_wb, exT_c,
                 s_w, d_w, s_wb, d_wb, s_c, d_c, zeros2d_hbm,
                 out_w, out_wb, out_c,
                 sbig_v, dbig_v, didxc_v, didxc2_v, exv_v, exv2_v,
                 rows_v, rows2_v, zbuf_v, osp, sem, sem2, sem3):
    cid_ax = lax.axis_index("c")
    sid_ax = lax.axis_index("s")
    half_base = cid_ax * _HALF
    zrb = jnp.minimum(sid_ax * _ORPT, _OSP_ROWS - _ORPT)
    orb = jnp.minimum(sid_ax * _ORPT, _HALF - _ORPT)

    def zero_osp():
        pltpu.sync_copy(zeros2d_hbm, zbuf_v)
        pltpu.sync_copy(zbuf_v, osp.at[pl.ds(zrb, _ORPT), :])

    zero_osp()
    plsc.subcore_barrier()

    for et, (xp_hbm, exT_hbm, s_e, d_e, out_hbm) in enumerate((
            (xp_a, exT_w, s_w, d_w, out_w),
            (xp_p, exT_wb, s_wb, d_wb, out_wb),
            (xp_p, exT_c, s_c, d_c, out_c))):

        def super_body(m, _, xp_hbm=xp_hbm, exT_hbm=exT_hbm,
                       s_e=s_e, d_e=d_e):
            sup = sid_ax + 16 * m

            @pl.when(sup < _NSUP)
            def _():
                base = sup * _SUPSZ
                pltpu.sync_copy(s_e.at[pl.ds(base, _SUPSZ)], sbig_v)
                pltpu.sync_copy(d_e.at[pl.ds(base, _SUPSZ)], dbig_v)
                rows_bufs = (rows_v, rows2_v)
                ex_bufs = (exv_v, exv2_v)
                dc_bufs = (didxc_v, didxc2_v)
                cp_rows = [None] * _SUP
                cp_ex = [None] * _SUP
                cp_sc = [None] * _SUP

                def fire(j):
                    b = j % 2
                    cp_rows[j] = pltpu.async_copy(
                        xp_hbm.at[sbig_v.at[pl.ds(j * _CHUNK, _CHUNK)]],
                        rows_bufs[b], sem2)
                    cp_ex[j] = pltpu.async_copy(
                        exT_hbm.at[sup * _SUP + j], ex_bufs[b], sem)

                fire(0)
                for j in range(_SUP):
                    b = j % 2
                    if j >= 1:
                        cp_sc[j - 1].wait()
                    if j + 1 < _SUP:
                        fire(j + 1)
                    cp_ex[j].wait()
                    cp_rows[j].wait()
                    for i in range(_CHUNK // 16):
                        d16 = dbig_v[pl.ds(j * _CHUNK + i * 16, 16)]
                        local = d16 - half_base
                        ok = (local >= 0) & (local < _HALF)
                        dc_bufs[b][pl.ds(i * 16, 16)] = jnp.where(
                            ok, local, _HALF)

                    def group_body(g, _, b=b):
                        wl = [ex_bufs[b][h, pl.ds(g * 16, 16)]
                              for h in range(H)]

                        def edge_body(e, _, b=b, wl=wl, g=g):
                            r = g * 16 + e
                            for h in range(H):
                                bc = _bcast_lane(wl[h], e)
                                rows_bufs[b][r, pl.ds(h * 16, 16)] = (
                                    rows_bufs[b][r, pl.ds(h * 16, 16)] * bc)
                            return 0

                        lax.fori_loop(0, 16, edge_body, 0)
                        return 0

                    lax.fori_loop(0, _CHUNK // 16, group_body, 0)
                    cp_sc[j] = pltpu.async_copy(
                        rows_bufs[b], osp.at[dc_bufs[b]], sem3, add=True)
                cp_sc[_SUP - 1].wait()
            return 0

        lax.fori_loop(0, (_NSUP + 15) // 16, super_body, 0)

        plsc.subcore_barrier()
        pltpu.sync_copy(osp.at[pl.ds(orb, _ORPT), :], zbuf_v)
        pltpu.sync_copy(zbuf_v, out_hbm.at[pl.ds(half_base + orb, _ORPT), :])
        plsc.subcore_barrier()
        if et < 2:
            zero_osp()
            plsc.subcore_barrier()


@jax.jit
def _pass_b(xp_a, xp_p, exT_w, exT_wb, exT_c,
            s_w, d_w, s_wb, d_wb, s_c, d_c, zeros2d):
    mesh = plsc.VectorSubcoreMesh(core_axis_name="c", subcore_axis_name="s")
    f = pl.kernel(
        _pass_b_body,
        mesh=mesh,
        out_type=[jax.ShapeDtypeStruct((N, HID), jnp.float32)] * 3,
        scratch_types=[
            pltpu.VMEM((_SUPSZ,), jnp.int32),
            pltpu.VMEM((_SUPSZ,), jnp.int32),
            pltpu.VMEM((_CHUNK,), jnp.int32),
            pltpu.VMEM((_CHUNK,), jnp.int32),
            pltpu.VMEM((H, _CHUNK), jnp.float32),
            pltpu.VMEM((H, _CHUNK), jnp.float32),
            pltpu.VMEM((_CHUNK, HID), jnp.float32),
            pltpu.VMEM((_CHUNK, HID), jnp.float32),
            pltpu.VMEM((_ORPT, HID), jnp.float32),
            pltpu.VMEM_SHARED((_OSP_ROWS, HID), jnp.float32),
            pltpu.SemaphoreType.DMA,
            pltpu.SemaphoreType.DMA,
            pltpu.SemaphoreType.DMA,
        ],
    )
    return f(xp_a, xp_p, exT_w, exT_wb, exT_c,
             s_w, d_w, s_wb, d_wb, s_c, d_c, zeros2d)


def _han_layer(x_dict, ei_dict, params, l):
    xp = {nt: _proj(x_dict[nt], params[f'proj_W_{nt}_{l}'],
                    params[f'proj_b_{nt}_{l}']).reshape(-1, H, DH)
          for nt in NODE_TYPES}

    a_src = {}
    a_dst = {}
    mx_rows = []
    for et, src_t, dst_t in EDGE_TYPES:
        asrc = (xp[src_t] * params[f'att_src_{et}_{l}']).sum(-1)
        adst = (xp[dst_t] * params[f'att_dst_{et}_{l}']).sum(-1)
        a_src[et] = asrc.T.reshape(-1)
        a_dst[et] = adst.T.reshape(-1)
        c_et = _lrelu(jnp.max(asrc) + jnp.max(adst))
        mx_rows.append(jnp.full((16,), c_et, jnp.float32))
    mx = jnp.concatenate(mx_rows)
    zeros = jnp.zeros((_N8,), jnp.float32)

    s = {et: ei_dict[et][0] for et, _, _ in EDGE_TYPES}
    d = {et: ei_dict[et][1] for et, _, _ in EDGE_TYPES}

    ex_w, ex_wb, ex_c, dp_w, dp_wb, dp_c = _pass_a(
        a_src['writes'], a_dst['writes'], a_src['written_by'],
        a_dst['written_by'], a_src['cites'], a_dst['cites'], mx, zeros,
        s['writes'], d['writes'], s['written_by'], d['written_by'],
        s['cites'], d['cites'])
    def rd_exp(dp):
        rd = 1.0 / (dp[:_N8] + dp[_N8:] + 1e-16)
        return jnp.repeat(rd.reshape(H, N).T, DH, axis=1)

    zeros2d = jnp.zeros((_ORPT, HID), jnp.float32)
    out_w, out_wb, out_c = _pass_b(
        xp['author'].reshape(N, HID), xp['paper'].reshape(N, HID),
        ex_w, ex_wb, ex_c,
        s['writes'], d['writes'], s['written_by'], d['written_by'],
        s['cites'], d['cites'], zeros2d)
    out_lists = {
        'author': [jax.nn.relu(out_wb * rd_exp(dp_wb))],
        'paper': [jax.nn.relu(out_w * rd_exp(dp_w)),
                  jax.nn.relu(out_c * rd_exp(dp_c))],
    }

    new_x = {}
    for nt in NODE_TYPES:
        outs = jnp.stack(out_lists[nt])
        kx = jnp.tanh(outs @ params[f'k_lin_W_{l}'] + params[f'k_lin_b_{l}']).mean(axis=1)
        score = (params[f'q_{l}'] * kx).sum(-1)
        attn = jax.nn.softmax(score)
        new_x[nt] = (attn[:, None, None] * outs).sum(0)
    return new_x


def kernel(x_author, x_paper, edge_index_writes, edge_index_written_by, edge_index_cites, params):
    x = {'author': x_author, 'paper': x_paper}
    ei = {'writes': edge_index_writes, 'written_by': edge_index_written_by, 'cites': edge_index_cites}
    for l in range(L):
        x = _han_layer(x, ei, params, l)
        if l < L - 1:
            x = {k: jax.nn.relu(v) for k, v in x.items()}
    return (x['author'], x['paper'])


# speedup vs baseline: 53.0909x; 1.0322x over previous
import functools

import jax
import jax.numpy as jnp
from jax import lax
from jax.experimental import pallas as pl
from jax.experimental.pallas import tpu as pltpu
from jax.experimental.pallas import tpu_sc as plsc

N = 10000
E = 320000
D_IN = 128
HID = 128
H = 8
DH = HID // H
L = 2
NODE_TYPES = ('author', 'paper')
EDGE_TYPES = (('writes', 'author', 'paper'), ('written_by', 'paper', 'author'), ('cites', 'paper', 'paper'))

_ROW_BLK = 400

_CHUNK = 128
_NCHUNK = E // _CHUNK
_NTILES = 32
_ROWS_PER_TILE = 632


def _proj_body(x_ref, w_ref, b_ref, o_ref):
    o_ref[...] = jnp.dot(x_ref[...], w_ref[...],
                         preferred_element_type=jnp.float32) + b_ref[...]


def _proj(x, w, b):
    m, k = x.shape
    n = w.shape[1]
    grid = m // _ROW_BLK
    return pl.pallas_call(
        _proj_body,
        grid=(grid,),
        in_specs=[
            pl.BlockSpec((_ROW_BLK, k), lambda i: (i, 0)),
            pl.BlockSpec((k, n), lambda i: (0, 0)),
            pl.BlockSpec((1, n), lambda i: (0, 0)),
        ],
        out_specs=pl.BlockSpec((_ROW_BLK, n), lambda i: (i, 0)),
        out_shape=jax.ShapeDtypeStruct((m, n), jnp.float32),
    )(x, w, b.reshape(1, n))


def _lrelu(v):
    return jnp.where(v >= 0.0, v, 0.2 * v)


_N8 = N * H
_DEN_PER_TILE = _N8 // 16
_SUPA = 2
_SUPA_SZ = _SUPA * _CHUNK
_NSUPA = E // _SUPA_SZ


def _pass_a_body(asrc_w, adst_w, asrc_wb, adst_wb, asrc_c, adst_c,
                 mx_hbm, zeros_hbm,
                 s_w, d_w, s_wb, d_wb, s_c, d_c,
                 ex_w, ex_wb, ex_c, dp_w, dp_wb, dp_c,
                 sbig_v, dbig_v, sidxh_v, sidxh2_v, didxh_v, didxh2_v,
                 sg_v, sg2_v, dg_v, dg2_v, exh_v, exh2_v, mx_v,
                 stage_v, dsp_w, dsp_wb, dsp_c, semg0, semg1, sem3):
    cid_ax = lax.axis_index("c")
    sid_ax = lax.axis_index("s")
    wid = sid_ax * 2 + cid_ax
    dbase = sid_ax * _DEN_PER_TILE

    pltpu.sync_copy(zeros_hbm.at[pl.ds(dbase, _DEN_PER_TILE)], stage_v)
    for dsp in (dsp_w, dsp_wb, dsp_c):
        pltpu.sync_copy(stage_v, dsp.at[pl.ds(dbase, _DEN_PER_TILE)])
    plsc.subcore_barrier()

    pltpu.sync_copy(mx_hbm, mx_v)
    si_bufs = (sidxh_v, sidxh2_v)
    di_bufs = (didxh_v, didxh2_v)
    sg_bufs = (sg_v, sg2_v)
    dg_bufs = (dg_v, dg2_v)
    ex_bufs = (exh_v, exh2_v)

    for et, (asrc, adst, s_e, d_e, ex_hbm, dsp) in enumerate((
            (asrc_w, adst_w, s_w, d_w, ex_w, dsp_w),
            (asrc_wb, adst_wb, s_wb, d_wb, ex_wb, dsp_wb),
            (asrc_c, adst_c, s_c, d_c, ex_c, dsp_c))):
        c_et = mx_v[pl.ds(et * 16, 16)]

        def super_body(m, _, asrc=asrc, adst=adst, s_e=s_e, d_e=d_e,
                       ex_hbm=ex_hbm, dsp=dsp, c_et=c_et):
            sup = wid + _NTILES * m

            @pl.when(sup < _NSUPA)
            def _():
                base = sup * _SUPA_SZ
                pltpu.sync_copy(s_e.at[pl.ds(base, _SUPA_SZ)], sbig_v)
                pltpu.sync_copy(d_e.at[pl.ds(base, _SUPA_SZ)], dbig_v)
                cp_g = [None] * _SUPA
                cp_sc = [None] * _SUPA

                def build_and_fire(j, asrc=asrc, adst=adst):
                    b = j % 2
                    semg = (semg0, semg1)[b]

                    for i in range(_CHUNK // 16):
                        s16 = sbig_v[pl.ds(j * _CHUNK + i * 16, 16)]
                        d16 = dbig_v[pl.ds(j * _CHUNK + i * 16, 16)]
                        for h in range(H):
                            si_bufs[b][h, pl.ds(i * 16, 16)] = s16 + (h * N)
                            di_bufs[b][h, pl.ds(i * 16, 16)] = d16 + (h * N)
                    cps = []
                    for h in range(H):
                        cps.append(pltpu.async_copy(
                            asrc.at[si_bufs[b].at[h]], sg_bufs[b].at[h], semg))
                        cps.append(pltpu.async_copy(
                            adst.at[di_bufs[b].at[h]], dg_bufs[b].at[h], semg))
                    cp_g[j] = cps

                for j in range(_SUPA):
                    b = j % 2
                    build_and_fire(j)
                    for cp in cp_g[j]:
                        cp.wait()

                    for h in range(H):
                        for i in range(_CHUNK // 16):
                            v = (sg_bufs[b][h, pl.ds(i * 16, 16)]
                                 + dg_bufs[b][h, pl.ds(i * 16, 16)])
                            ex_bufs[b][h, pl.ds(i * 16, 16)] = jnp.exp(
                                _lrelu(v) - c_et)
                    pltpu.sync_copy(ex_bufs[b], ex_hbm.at[sup * _SUPA + j])
                    for h in range(H):
                        pltpu.sync_copy(ex_bufs[b].at[h],
                                        dsp.at[di_bufs[b].at[h]], add=True)
            return 0

        lax.fori_loop(0, (_NSUPA + _NTILES - 1) // _NTILES, super_body, 0)

    plsc.subcore_barrier()
    for dsp, dp in ((dsp_w, dp_w), (dsp_wb, dp_wb), (dsp_c, dp_c)):
        pltpu.sync_copy(dsp.at[pl.ds(dbase, _DEN_PER_TILE)], stage_v)
        pltpu.sync_copy(stage_v,
                        dp.at[pl.ds(cid_ax * _N8 + dbase, _DEN_PER_TILE)])


@jax.jit
def _pass_a(asrc_w, adst_w, asrc_wb, adst_wb, asrc_c, adst_c, mx, zeros,
            s_w, d_w, s_wb, d_wb, s_c, d_c):
    mesh = plsc.VectorSubcoreMesh(core_axis_name="c", subcore_axis_name="s")
    f = pl.kernel(
        _pass_a_body,
        mesh=mesh,
        out_type=[jax.ShapeDtypeStruct((_NCHUNK, H, _CHUNK), jnp.float32)] * 3
                 + [jax.ShapeDtypeStruct((2 * _N8,), jnp.float32)] * 3,
        scratch_types=[
            pltpu.VMEM((_SUPA_SZ,), jnp.int32),
            pltpu.VMEM((_SUPA_SZ,), jnp.int32),
            pltpu.VMEM((H, _CHUNK), jnp.int32),
            pltpu.VMEM((H, _CHUNK), jnp.int32),
            pltpu.VMEM((H, _CHUNK), jnp.int32),
            pltpu.VMEM((H, _CHUNK), jnp.int32),
            pltpu.VMEM((H, _CHUNK), jnp.float32),
            pltpu.VMEM((H, _CHUNK), jnp.float32),
            pltpu.VMEM((H, _CHUNK), jnp.float32),
            pltpu.VMEM((H, _CHUNK), jnp.float32),
            pltpu.VMEM((H, _CHUNK), jnp.float32),
            pltpu.VMEM((H, _CHUNK), jnp.float32),
            pltpu.VMEM((48,), jnp.float32),
            pltpu.VMEM((_DEN_PER_TILE,), jnp.float32),
            pltpu.VMEM_SHARED((_N8,), jnp.float32),
            pltpu.VMEM_SHARED((_N8,), jnp.float32),
            pltpu.VMEM_SHARED((_N8,), jnp.float32),
            pltpu.SemaphoreType.DMA,
            pltpu.SemaphoreType.DMA,
            pltpu.SemaphoreType.DMA,
        ],
    )
    return f(asrc_w, adst_w, asrc_wb, adst_wb, asrc_c, adst_c, mx, zeros,
             s_w, d_w, s_wb, d_wb, s_c, d_c)


_HALF = N // 2
_OSP_ROWS = _HALF + 8
_ORPT = 320
_SUPB = 10
_SUPB_SZ = _SUPB * _CHUNK
_NSUPB = E // _SUPB_SZ


def _bcast_lane(v, lane):
    idx = jnp.full((16, 1), lane, jnp.int32)
    dn = lax.GatherDimensionNumbers(offset_dims=(), collapsed_slice_dims=(0,),
                                    start_index_map=(0,))
    return lax.gather(v, idx, dn, (1,),
                      mode=lax.GatherScatterMode.PROMISE_IN_BOUNDS)


def _pass_b_body(xp_a, xp_p, exT_w, exT_wb, exT_c,
                 s_w, d_w, s_wb, d_wb, s_c, d_c, zeros2d_hbm,
                 out_w, out_wb, out_c,
                 sbig_v, dbig_v, didxc_v, didxc2_v, exv_v, exv2_v,
                 rows_v, rows2_v, zbuf_v, osp, semp0, semp1, sem3):
    cid_ax = lax.axis_index("c")
    sid_ax = lax.axis_index("s")
    half_base = cid_ax * _HALF
    zrb = jnp.minimum(sid_ax * _ORPT, _OSP_ROWS - _ORPT)
    orb = jnp.minimum(sid_ax * _ORPT, _HALF - _ORPT)

    def zero_osp():
        pltpu.sync_copy(zeros2d_hbm, zbuf_v)
        pltpu.sync_copy(zbuf_v, osp.at[pl.ds(zrb, _ORPT), :])

    zero_osp()
    plsc.subcore_barrier()

    for et, (xp_hbm, exT_hbm, s_e, d_e, out_hbm) in enumerate((
            (xp_a, exT_w, s_w, d_w, out_w),
            (xp_p, exT_wb, s_wb, d_wb, out_wb),
            (xp_p, exT_c, s_c, d_c, out_c))):

        def super_body(m, _, xp_hbm=xp_hbm, exT_hbm=exT_hbm,
                       s_e=s_e, d_e=d_e):
            sup = sid_ax + 16 * m

            @pl.when(sup < _NSUPB)
            def _():
                base = sup * _SUPB_SZ
                pltpu.sync_copy(s_e.at[pl.ds(base, _SUPB_SZ)], sbig_v)
                pltpu.sync_copy(d_e.at[pl.ds(base, _SUPB_SZ)], dbig_v)
                rows_bufs = (rows_v, rows2_v)
                ex_bufs = (exv_v, exv2_v)
                dc_bufs = (didxc_v, didxc2_v)
                cp_rows = [None] * _SUPB
                cp_ex = [None] * _SUPB
                cp_sc = [None] * _SUPB

                def fire(j):
                    b = j % 2
                    semp = (semp0, semp1)[b]
                    cp_rows[j] = pltpu.async_copy(
                        xp_hbm.at[sbig_v.at[pl.ds(j * _CHUNK, _CHUNK)]],
                        rows_bufs[b], semp)
                    cp_ex[j] = pltpu.async_copy(
                        exT_hbm.at[sup * _SUPB + j], ex_bufs[b], semp)

                fire(0)
                for j in range(_SUPB):
                    b = j % 2
                    if j >= 1:
                        cp_sc[j - 1].wait()
                    if j + 1 < _SUPB:
                        fire(j + 1)
                    cp_ex[j].wait()
                    cp_rows[j].wait()
                    for i in range(_CHUNK // 16):
                        d16 = dbig_v[pl.ds(j * _CHUNK + i * 16, 16)]
                        local = d16 - half_base
                        ok = (local >= 0) & (local < _HALF)
                        dc_bufs[b][pl.ds(i * 16, 16)] = jnp.where(
                            ok, local, _HALF)

                    def group_body(g, _, b=b):
                        wl = [ex_bufs[b][h, pl.ds(g * 16, 16)]
                              for h in range(H)]

                        def edge_body(e, _, b=b, wl=wl, g=g):
                            r = g * 16 + e
                            for h in range(H):
                                bc = _bcast_lane(wl[h], e)
                                rows_bufs[b][r, pl.ds(h * 16, 16)] = (
                                    rows_bufs[b][r, pl.ds(h * 16, 16)] * bc)
                            return 0

                        lax.fori_loop(0, 16, edge_body, 0)
                        return 0

                    lax.fori_loop(0, _CHUNK // 16, group_body, 0)
                    cp_sc[j] = pltpu.async_copy(
                        rows_bufs[b], osp.at[dc_bufs[b]], sem3, add=True)
                cp_sc[_SUPB - 1].wait()
            return 0

        lax.fori_loop(0, (_NSUPB + 15) // 16, super_body, 0)

        plsc.subcore_barrier()
        pltpu.sync_copy(osp.at[pl.ds(orb, _ORPT), :], zbuf_v)
        pltpu.sync_copy(zbuf_v, out_hbm.at[pl.ds(half_base + orb, _ORPT), :])
        plsc.subcore_barrier()
        if et < 2:
            zero_osp()
            plsc.subcore_barrier()


@jax.jit
def _pass_b(xp_a, xp_p, exT_w, exT_wb, exT_c,
            s_w, d_w, s_wb, d_wb, s_c, d_c, zeros2d):
    mesh = plsc.VectorSubcoreMesh(core_axis_name="c", subcore_axis_name="s")
    f = pl.kernel(
        _pass_b_body,
        mesh=mesh,
        out_type=[jax.ShapeDtypeStruct((N, HID), jnp.float32)] * 3,
        scratch_types=[
            pltpu.VMEM((_SUPB_SZ,), jnp.int32),
            pltpu.VMEM((_SUPB_SZ,), jnp.int32),
            pltpu.VMEM((_CHUNK,), jnp.int32),
            pltpu.VMEM((_CHUNK,), jnp.int32),
            pltpu.VMEM((H, _CHUNK), jnp.float32),
            pltpu.VMEM((H, _CHUNK), jnp.float32),
            pltpu.VMEM((_CHUNK, HID), jnp.float32),
            pltpu.VMEM((_CHUNK, HID), jnp.float32),
            pltpu.VMEM((_ORPT, HID), jnp.float32),
            pltpu.VMEM_SHARED((_OSP_ROWS, HID), jnp.float32),
            pltpu.SemaphoreType.DMA,
            pltpu.SemaphoreType.DMA,
            pltpu.SemaphoreType.DMA,
        ],
    )
    return f(xp_a, xp_p, exT_w, exT_wb, exT_c,
             s_w, d_w, s_wb, d_wb, s_c, d_c, zeros2d)


def _han_layer(x_dict, ei_dict, params, l):
    xp = {nt: _proj(x_dict[nt], params[f'proj_W_{nt}_{l}'],
                    params[f'proj_b_{nt}_{l}']).reshape(-1, H, DH)
          for nt in NODE_TYPES}

    a_src = {}
    a_dst = {}
    mx_rows = []
    for et, src_t, dst_t in EDGE_TYPES:
        asrc = (xp[src_t] * params[f'att_src_{et}_{l}']).sum(-1)
        adst = (xp[dst_t] * params[f'att_dst_{et}_{l}']).sum(-1)
        a_src[et] = asrc.T.reshape(-1)
        a_dst[et] = adst.T.reshape(-1)
        c_et = _lrelu(jnp.max(asrc) + jnp.max(adst))
        mx_rows.append(jnp.full((16,), c_et, jnp.float32))
    mx = jnp.concatenate(mx_rows)
    zeros = jnp.zeros((_N8,), jnp.float32)

    s = {et: ei_dict[et][0] for et, _, _ in EDGE_TYPES}
    d = {et: ei_dict[et][1] for et, _, _ in EDGE_TYPES}

    ex_w, ex_wb, ex_c, dp_w, dp_wb, dp_c = _pass_a(
        a_src['writes'], a_dst['writes'], a_src['written_by'],
        a_dst['written_by'], a_src['cites'], a_dst['cites'], mx, zeros,
        s['writes'], d['writes'], s['written_by'], d['written_by'],
        s['cites'], d['cites'])
    def rd_exp(dp):
        rd = 1.0 / (dp[:_N8] + dp[_N8:] + 1e-16)
        return jnp.repeat(rd.reshape(H, N).T, DH, axis=1)

    zeros2d = jnp.zeros((_ORPT, HID), jnp.float32)
    out_w, out_wb, out_c = _pass_b(
        xp['author'].reshape(N, HID), xp['paper'].reshape(N, HID),
        ex_w, ex_wb, ex_c,
        s['writes'], d['writes'], s['written_by'], d['written_by'],
        s['cites'], d['cites'], zeros2d)
    out_lists = {
        'author': [jax.nn.relu(out_wb * rd_exp(dp_wb))],
        'paper': [jax.nn.relu(out_w * rd_exp(dp_w)),
                  jax.nn.relu(out_c * rd_exp(dp_c))],
    }

    new_x = {}
    for nt in NODE_TYPES:
        outs = jnp.stack(out_lists[nt])
        kx = jnp.tanh(outs @ params[f'k_lin_W_{l}'] + params[f'k_lin_b_{l}']).mean(axis=1)
        score = (params[f'q_{l}'] * kx).sum(-1)
        attn = jax.nn.softmax(score)
        new_x[nt] = (attn[:, None, None] * outs).sum(0)
    return new_x


def kernel(x_author, x_paper, edge_index_writes, edge_index_written_by, edge_index_cites, params):
    x = {'author': x_author, 'paper': x_paper}
    ei = {'writes': edge_index_writes, 'written_by': edge_index_written_by, 'cites': edge_index_cites}
    for l in range(L):
        x = _han_layer(x, ei, params, l)
        if l < L - 1:
            x = {k: jax.nn.relu(v) for k, v in x.items()}
    return (x['author'], x['paper'])


# speedup vs baseline: 63.0160x; 1.1869x over previous
import functools

import jax
import jax.numpy as jnp
from jax import lax
from jax.experimental import pallas as pl
from jax.experimental.pallas import tpu as pltpu
from jax.experimental.pallas import tpu_sc as plsc

N = 10000
E = 320000
D_IN = 128
HID = 128
H = 8
DH = HID // H
L = 2
NODE_TYPES = ('author', 'paper')
EDGE_TYPES = (('writes', 'author', 'paper'), ('written_by', 'paper', 'author'), ('cites', 'paper', 'paper'))

_ROW_BLK = 400

_CHUNK = 128
_NCHUNK = E // _CHUNK
_NTILES = 32
_ROWS_PER_TILE = 632


def _proj_body(x_ref, w_ref, b_ref, o_ref):
    o_ref[...] = jnp.dot(x_ref[...], w_ref[...],
                         preferred_element_type=jnp.float32) + b_ref[...]


def _proj(x, w, b):
    m, k = x.shape
    n = w.shape[1]
    grid = m // _ROW_BLK
    return pl.pallas_call(
        _proj_body,
        grid=(grid,),
        in_specs=[
            pl.BlockSpec((_ROW_BLK, k), lambda i: (i, 0)),
            pl.BlockSpec((k, n), lambda i: (0, 0)),
            pl.BlockSpec((1, n), lambda i: (0, 0)),
        ],
        out_specs=pl.BlockSpec((_ROW_BLK, n), lambda i: (i, 0)),
        out_shape=jax.ShapeDtypeStruct((m, n), jnp.float32),
    )(x, w, b.reshape(1, n))


def _lrelu(v):
    return jnp.where(v >= 0.0, v, 0.2 * v)


_N8 = N * H
_DEN_PER_TILE = _N8 // 16
_SUPA = 2
_SUPA_SZ = _SUPA * _CHUNK
_NSUPA = E // _SUPA_SZ


def _pass_a_body(asrc_w, adst_w, asrc_wb, adst_wb, asrc_c, adst_c,
                 mx_hbm, zeros_hbm,
                 s_w, d_w, s_wb, d_wb, s_c, d_c,
                 ex_w, ex_wb, ex_c, dp_w, dp_wb, dp_c,
                 sbig_v, dbig_v, sidxh_v, sidxh2_v, didxh_v, didxh2_v,
                 sg_v, sg2_v, dg_v, dg2_v, exh_v, exh2_v, mx_v,
                 stage_v, dsp_w, dsp_wb, dsp_c, semg0, semg1, sem3):
    cid_ax = lax.axis_index("c")
    sid_ax = lax.axis_index("s")
    wid = sid_ax * 2 + cid_ax
    dbase = sid_ax * _DEN_PER_TILE

    pltpu.sync_copy(zeros_hbm.at[pl.ds(dbase, _DEN_PER_TILE)], stage_v)
    for dsp in (dsp_w, dsp_wb, dsp_c):
        pltpu.sync_copy(stage_v, dsp.at[pl.ds(dbase, _DEN_PER_TILE)])
    plsc.subcore_barrier()

    pltpu.sync_copy(mx_hbm, mx_v)
    si_bufs = (sidxh_v, sidxh2_v)
    di_bufs = (didxh_v, didxh2_v)
    sg_bufs = (sg_v, sg2_v)
    dg_bufs = (dg_v, dg2_v)
    ex_bufs = (exh_v, exh2_v)

    for et, (asrc, adst, s_e, d_e, ex_hbm, dsp) in enumerate((
            (asrc_w, adst_w, s_w, d_w, ex_w, dsp_w),
            (asrc_wb, adst_wb, s_wb, d_wb, ex_wb, dsp_wb),
            (asrc_c, adst_c, s_c, d_c, ex_c, dsp_c))):
        c_et = mx_v[pl.ds(et * 16, 16)]

        def super_body(m, _, asrc=asrc, adst=adst, s_e=s_e, d_e=d_e,
                       ex_hbm=ex_hbm, dsp=dsp, c_et=c_et):
            sup = wid + _NTILES * m

            @pl.when(sup < _NSUPA)
            def _():
                base = sup * _SUPA_SZ
                pltpu.sync_copy(s_e.at[pl.ds(base, _SUPA_SZ)], sbig_v)
                pltpu.sync_copy(d_e.at[pl.ds(base, _SUPA_SZ)], dbig_v)
                cp_g = [None] * _SUPA
                cp_sc = [None] * _SUPA

                def build_and_fire(j, asrc=asrc, adst=adst):
                    b = j % 2
                    semg = (semg0, semg1)[b]

                    for i in range(_CHUNK // 16):
                        s16 = sbig_v[pl.ds(j * _CHUNK + i * 16, 16)]
                        d16 = dbig_v[pl.ds(j * _CHUNK + i * 16, 16)]
                        for h in range(H):
                            si_bufs[b][h, pl.ds(i * 16, 16)] = s16 + (h * N)
                            di_bufs[b][h, pl.ds(i * 16, 16)] = d16 + (h * N)
                    cps = []
                    for h in range(H):
                        cps.append(pltpu.async_copy(
                            asrc.at[si_bufs[b].at[h]], sg_bufs[b].at[h], semg))
                        cps.append(pltpu.async_copy(
                            adst.at[di_bufs[b].at[h]], dg_bufs[b].at[h], semg))
                    cp_g[j] = cps

                build_and_fire(0)
                for j in range(_SUPA):
                    b = j % 2
                    if j >= 1:
                        for cp in cp_sc[j - 1]:
                            cp.wait()
                    if j + 1 < _SUPA:
                        build_and_fire(j + 1)
                    for cp in cp_g[j]:
                        cp.wait()

                    for h in range(H):
                        for i in range(_CHUNK // 16):
                            v = (sg_bufs[b][h, pl.ds(i * 16, 16)]
                                 + dg_bufs[b][h, pl.ds(i * 16, 16)])
                            ex_bufs[b][h, pl.ds(i * 16, 16)] = jnp.exp(
                                _lrelu(v) - c_et)
                    pltpu.sync_copy(ex_bufs[b], ex_hbm.at[sup * _SUPA + j])
                    cp_sc[j] = [
                        pltpu.async_copy(ex_bufs[b].at[h],
                                         dsp.at[di_bufs[b].at[h]],
                                         sem3, add=True)
                        for h in range(H)]
                for cp in cp_sc[_SUPA - 1]:
                    cp.wait()
            return 0

        lax.fori_loop(0, (_NSUPA + _NTILES - 1) // _NTILES, super_body, 0)

    plsc.subcore_barrier()
    for dsp, dp in ((dsp_w, dp_w), (dsp_wb, dp_wb), (dsp_c, dp_c)):
        pltpu.sync_copy(dsp.at[pl.ds(dbase, _DEN_PER_TILE)], stage_v)
        pltpu.sync_copy(stage_v,
                        dp.at[pl.ds(cid_ax * _N8 + dbase, _DEN_PER_TILE)])


@jax.jit
def _pass_a(asrc_w, adst_w, asrc_wb, adst_wb, asrc_c, adst_c, mx, zeros,
            s_w, d_w, s_wb, d_wb, s_c, d_c):
    mesh = plsc.VectorSubcoreMesh(core_axis_name="c", subcore_axis_name="s")
    f = pl.kernel(
        _pass_a_body,
        mesh=mesh,
        out_type=[jax.ShapeDtypeStruct((_NCHUNK, H, _CHUNK), jnp.float32)] * 3
                 + [jax.ShapeDtypeStruct((2 * _N8,), jnp.float32)] * 3,
        scratch_types=[
            pltpu.VMEM((_SUPA_SZ,), jnp.int32),
            pltpu.VMEM((_SUPA_SZ,), jnp.int32),
            pltpu.VMEM((H, _CHUNK), jnp.int32),
            pltpu.VMEM((H, _CHUNK), jnp.int32),
            pltpu.VMEM((H, _CHUNK), jnp.int32),
            pltpu.VMEM((H, _CHUNK), jnp.int32),
            pltpu.VMEM((H, _CHUNK), jnp.float32),
            pltpu.VMEM((H, _CHUNK), jnp.float32),
            pltpu.VMEM((H, _CHUNK), jnp.float32),
            pltpu.VMEM((H, _CHUNK), jnp.float32),
            pltpu.VMEM((H, _CHUNK), jnp.float32),
            pltpu.VMEM((H, _CHUNK), jnp.float32),
            pltpu.VMEM((48,), jnp.float32),
            pltpu.VMEM((_DEN_PER_TILE,), jnp.float32),
            pltpu.VMEM_SHARED((_N8,), jnp.float32),
            pltpu.VMEM_SHARED((_N8,), jnp.float32),
            pltpu.VMEM_SHARED((_N8,), jnp.float32),
            pltpu.SemaphoreType.DMA,
            pltpu.SemaphoreType.DMA,
            pltpu.SemaphoreType.DMA,
        ],
    )
    return f(asrc_w, adst_w, asrc_wb, adst_wb, asrc_c, adst_c, mx, zeros,
             s_w, d_w, s_wb, d_wb, s_c, d_c)


_HALF = N // 2
_OSP_ROWS = _HALF + 8
_ORPT = 320
_SUPB = 10
_SUPB_SZ = _SUPB * _CHUNK
_NSUPB = E // _SUPB_SZ


def _bcast_lane(v, lane):
    idx = jnp.full((16, 1), lane, jnp.int32)
    dn = lax.GatherDimensionNumbers(offset_dims=(), collapsed_slice_dims=(0,),
                                    start_index_map=(0,))
    return lax.gather(v, idx, dn, (1,),
                      mode=lax.GatherScatterMode.PROMISE_IN_BOUNDS)


def _pass_b_body(xp_a, xp_p, exT_w, exT_wb, exT_c,
                 s_w, d_w, s_wb, d_wb, s_c, d_c, zeros2d_hbm,
                 out_w, out_wb, out_c,
                 sbig_v, dbig_v, didxc_v, didxc2_v, exv_v, exv2_v,
                 rows_v, rows2_v, zbuf_v, osp, semp0, semp1, sem3):
    cid_ax = lax.axis_index("c")
    sid_ax = lax.axis_index("s")
    half_base = cid_ax * _HALF
    zrb = jnp.minimum(sid_ax * _ORPT, _OSP_ROWS - _ORPT)
    orb = jnp.minimum(sid_ax * _ORPT, _HALF - _ORPT)

    def zero_osp():
        pltpu.sync_copy(zeros2d_hbm, zbuf_v)
        pltpu.sync_copy(zbuf_v, osp.at[pl.ds(zrb, _ORPT), :])

    zero_osp()
    plsc.subcore_barrier()

    for et, (xp_hbm, exT_hbm, s_e, d_e, out_hbm) in enumerate((
            (xp_a, exT_w, s_w, d_w, out_w),
            (xp_p, exT_wb, s_wb, d_wb, out_wb),
            (xp_p, exT_c, s_c, d_c, out_c))):

        def super_body(m, _, xp_hbm=xp_hbm, exT_hbm=exT_hbm,
                       s_e=s_e, d_e=d_e):
            sup = sid_ax + 16 * m

            @pl.when(sup < _NSUPB)
            def _():
                base = sup * _SUPB_SZ
                pltpu.sync_copy(s_e.at[pl.ds(base, _SUPB_SZ)], sbig_v)
                pltpu.sync_copy(d_e.at[pl.ds(base, _SUPB_SZ)], dbig_v)
                rows_bufs = (rows_v, rows2_v)
                ex_bufs = (exv_v, exv2_v)
                dc_bufs = (didxc_v, didxc2_v)
                cp_rows = [None] * _SUPB
                cp_ex = [None] * _SUPB
                cp_sc = [None] * _SUPB

                def fire(j):
                    b = j % 2
                    semp = (semp0, semp1)[b]
                    cp_rows[j] = pltpu.async_copy(
                        xp_hbm.at[sbig_v.at[pl.ds(j * _CHUNK, _CHUNK)]],
                        rows_bufs[b], semp)
                    cp_ex[j] = pltpu.async_copy(
                        exT_hbm.at[sup * _SUPB + j], ex_bufs[b], semp)

                fire(0)
                for j in range(_SUPB):
                    b = j % 2
                    if j >= 1:
                        cp_sc[j - 1].wait()
                    if j + 1 < _SUPB:
                        fire(j + 1)
                    cp_ex[j].wait()
                    cp_rows[j].wait()
                    for i in range(_CHUNK // 16):
                        d16 = dbig_v[pl.ds(j * _CHUNK + i * 16, 16)]
                        local = d16 - half_base
                        ok = (local >= 0) & (local < _HALF)
                        dc_bufs[b][pl.ds(i * 16, 16)] = jnp.where(
                            ok, local, _HALF)

                    def group_body(g, _, b=b):
                        wl = [ex_bufs[b][h, pl.ds(g * 16, 16)]
                              for h in range(H)]

                        def edge_body(e, _, b=b, wl=wl, g=g):
                            r = g * 16 + e
                            for h in range(H):
                                bc = _bcast_lane(wl[h], e)
                                rows_bufs[b][r, pl.ds(h * 16, 16)] = (
                                    rows_bufs[b][r, pl.ds(h * 16, 16)] * bc)
                            return 0

                        lax.fori_loop(0, 16, edge_body, 0)
                        return 0

                    lax.fori_loop(0, _CHUNK // 16, group_body, 0)
                    cp_sc[j] = pltpu.async_copy(
                        rows_bufs[b], osp.at[dc_bufs[b]], sem3, add=True)
                cp_sc[_SUPB - 1].wait()
            return 0

        lax.fori_loop(0, (_NSUPB + 15) // 16, super_body, 0)

        plsc.subcore_barrier()
        pltpu.sync_copy(osp.at[pl.ds(orb, _ORPT), :], zbuf_v)
        pltpu.sync_copy(zbuf_v, out_hbm.at[pl.ds(half_base + orb, _ORPT), :])
        plsc.subcore_barrier()
        if et < 2:
            zero_osp()
            plsc.subcore_barrier()


@jax.jit
def _pass_b(xp_a, xp_p, exT_w, exT_wb, exT_c,
            s_w, d_w, s_wb, d_wb, s_c, d_c, zeros2d):
    mesh = plsc.VectorSubcoreMesh(core_axis_name="c", subcore_axis_name="s")
    f = pl.kernel(
        _pass_b_body,
        mesh=mesh,
        out_type=[jax.ShapeDtypeStruct((N, HID), jnp.float32)] * 3,
        scratch_types=[
            pltpu.VMEM((_SUPB_SZ,), jnp.int32),
            pltpu.VMEM((_SUPB_SZ,), jnp.int32),
            pltpu.VMEM((_CHUNK,), jnp.int32),
            pltpu.VMEM((_CHUNK,), jnp.int32),
            pltpu.VMEM((H, _CHUNK), jnp.float32),
            pltpu.VMEM((H, _CHUNK), jnp.float32),
            pltpu.VMEM((_CHUNK, HID), jnp.float32),
            pltpu.VMEM((_CHUNK, HID), jnp.float32),
            pltpu.VMEM((_ORPT, HID), jnp.float32),
            pltpu.VMEM_SHARED((_OSP_ROWS, HID), jnp.float32),
            pltpu.SemaphoreType.DMA,
            pltpu.SemaphoreType.DMA,
            pltpu.SemaphoreType.DMA,
        ],
    )
    return f(xp_a, xp_p, exT_w, exT_wb, exT_c,
             s_w, d_w, s_wb, d_wb, s_c, d_c, zeros2d)


def _han_layer(x_dict, ei_dict, params, l):
    xp = {nt: _proj(x_dict[nt], params[f'proj_W_{nt}_{l}'],
                    params[f'proj_b_{nt}_{l}']).reshape(-1, H, DH)
          for nt in NODE_TYPES}

    a_src = {}
    a_dst = {}
    mx_rows = []
    for et, src_t, dst_t in EDGE_TYPES:
        asrc = (xp[src_t] * params[f'att_src_{et}_{l}']).sum(-1)
        adst = (xp[dst_t] * params[f'att_dst_{et}_{l}']).sum(-1)
        a_src[et] = asrc.T.reshape(-1)
        a_dst[et] = adst.T.reshape(-1)
        c_et = _lrelu(jnp.max(asrc) + jnp.max(adst))
        mx_rows.append(jnp.full((16,), c_et, jnp.float32))
    mx = jnp.concatenate(mx_rows)
    zeros = jnp.zeros((_N8,), jnp.float32)

    s = {et: ei_dict[et][0] for et, _, _ in EDGE_TYPES}
    d = {et: ei_dict[et][1] for et, _, _ in EDGE_TYPES}

    ex_w, ex_wb, ex_c, dp_w, dp_wb, dp_c = _pass_a(
        a_src['writes'], a_dst['writes'], a_src['written_by'],
        a_dst['written_by'], a_src['cites'], a_dst['cites'], mx, zeros,
        s['writes'], d['writes'], s['written_by'], d['written_by'],
        s['cites'], d['cites'])
    def rd_exp(dp):
        rd = 1.0 / (dp[:_N8] + dp[_N8:] + 1e-16)
        return jnp.repeat(rd.reshape(H, N).T, DH, axis=1)

    zeros2d = jnp.zeros((_ORPT, HID), jnp.float32)
    out_w, out_wb, out_c = _pass_b(
        xp['author'].reshape(N, HID), xp['paper'].reshape(N, HID),
        ex_w, ex_wb, ex_c,
        s['writes'], d['writes'], s['written_by'], d['written_by'],
        s['cites'], d['cites'], zeros2d)
    out_lists = {
        'author': [jax.nn.relu(out_wb * rd_exp(dp_wb))],
        'paper': [jax.nn.relu(out_w * rd_exp(dp_w)),
                  jax.nn.relu(out_c * rd_exp(dp_c))],
    }

    new_x = {}
    for nt in NODE_TYPES:
        outs = jnp.stack(out_lists[nt])
        kx = jnp.tanh(outs @ params[f'k_lin_W_{l}'] + params[f'k_lin_b_{l}']).mean(axis=1)
        score = (params[f'q_{l}'] * kx).sum(-1)
        attn = jax.nn.softmax(score)
        new_x[nt] = (attn[:, None, None] * outs).sum(0)
    return new_x


def kernel(x_author, x_paper, edge_index_writes, edge_index_written_by, edge_index_cites, params):
    x = {'author': x_author, 'paper': x_paper}
    ei = {'writes': edge_index_writes, 'written_by': edge_index_written_by, 'cites': edge_index_cites}
    for l in range(L):
        x = _han_layer(x, ei, params, l)
        if l < L - 1:
            x = {k: jax.nn.relu(v) for k, v in x.items()}
    return (x['author'], x['paper'])
